# Initial kernel scaffold; baseline (speedup 1.0000x reference)
import jax
import jax.numpy as jnp
from jax.experimental import pallas as pl


def kernel(x_temporal, edge_weight, Wx, bx, Wh, bh, W1, b1, W2, b2, W3, b3, edge_index, batch):
    raise NotImplementedError("write your pallas kernel here")


# speedup vs baseline: 3.1392x; 3.1392x over previous
import functools

import jax
import jax.numpy as jnp
from jax import lax
from jax.experimental import pallas as pl
from jax.experimental.pallas import tpu as pltpu
from jax.experimental.pallas import tpu_sc as plsc

NC = 2
NS = 16
NW = NC * NS
LANE = 128
F32 = jnp.float32


def _mesh():
    return plsc.VectorSubcoreMesh(core_axis_name="c", subcore_axis_name="s")


@functools.lru_cache(maxsize=None)
def _deg_kernel(N, NCHG):
    rows = N // NS
    NCHH = NCHG // 2

    @functools.partial(
        pl.kernel,
        out_type=jax.ShapeDtypeStruct((NC, N, LANE), F32),
        mesh=_mesh(),
        compiler_params=pltpu.CompilerParams(needs_layout_passes=False),
        scratch_types=[
            pltpu.VMEM((NCHH, LANE), jnp.int32),
            pltpu.VMEM((NCHH * LANE,), F32),
            pltpu.VMEM((LANE, LANE), F32),
            pltpu.VMEM((LANE, LANE), F32),
            pltpu.VMEM_SHARED((N, LANE), F32),
            pltpu.SemaphoreType.DMA,
            pltpu.SemaphoreType.DMA,
        ],
    )
    def k(srcr, ewf, z128, out, src_v, w_v, rb0, rb1, acc, s0, s1):
        bufs = [rb0, rb1]
        ss = [s0, s1]
        c = lax.axis_index("c")
        s = lax.axis_index("s")
        w = c * NS + s
        pltpu.sync_copy(z128.at[pl.ds(s * rows, rows)],
                        acc.at[pl.ds(s * rows, rows)])
        iota = lax.iota(jnp.int32, 16)
        zcol = iota * 0
        plsc.subcore_barrier()

        def build(j, b):
            for g in range(8):
                wv = w_v[pl.ds(j * LANE + g * 16, 16)]
                plsc.store_scatter(bufs[b], [g * 16 + iota, zcol], wv)

        def s_start(j, b):
            pltpu.async_copy(bufs[b], acc.at[src_v.at[j]], ss[b], add=True)

        def s_wait(j, b):
            pltpu.make_async_copy(bufs[b], acc.at[src_v.at[j]], ss[b]).wait()

        def step(j, b, with_wait):
            if with_wait:
                s_wait(j - 2, b)
            build(j, b)
            s_start(j, b)

        for half in range(2):
            pltpu.sync_copy(srcr.at[w, pl.ds(half * NCHH, NCHH)], src_v)
            pltpu.sync_copy(ewf.at[w, pl.ds(half * NCHH * LANE, NCHH * LANE)],
                            w_v)
            step(0, 0, False)
            step(1, 1, False)

            def pair(jj, carry):
                for b in range(2):
                    step(jj * 2 + b, b, True)
                return carry

            lax.fori_loop(1, NCHH // 2, pair, 0)
            s_wait(NCHH - 2, 0)
            s_wait(NCHH - 1, 1)
        plsc.subcore_barrier()
        pltpu.sync_copy(acc.at[pl.ds(s * rows, rows)],
                        out.at[c, pl.ds(s * rows, rows)])

    return k


@functools.lru_cache(maxsize=None)
def _norm_kernel(N, NCHG):
    @functools.partial(
        pl.kernel,
        out_type=jax.ShapeDtypeStruct((NW, NCHG * LANE), F32),
        mesh=_mesh(),
        compiler_params=pltpu.CompilerParams(needs_layout_passes=False),
        scratch_types=[
            pltpu.VMEM((NCHG * LANE,), jnp.int32),
            pltpu.VMEM((NCHG * LANE,), jnp.int32),
            pltpu.VMEM((NCHG * LANE,), F32),
            pltpu.VMEM((N,), F32),
            pltpu.VMEM((NCHG * LANE,), F32),
        ],
    )
    def k(srcf, dstf, ewf, dis, out, src_v, dst_v, w_v, dis_v, nrm_v):
        c = lax.axis_index("c")
        s = lax.axis_index("s")
        w = c * NS + s
        pltpu.sync_copy(srcf.at[w], src_v)
        pltpu.sync_copy(dstf.at[w], dst_v)
        pltpu.sync_copy(ewf.at[w], w_v)
        pltpu.sync_copy(dis, dis_v)

        def chunk(j, carry):
            for g in range(8):
                o = j * LANE + g * 16
                si = src_v[pl.ds(o, 16)]
                di = dst_v[pl.ds(o, 16)]
                dsv = plsc.load_gather(dis_v, [si])
                ddv = plsc.load_gather(dis_v, [di])
                nrm_v[pl.ds(o, 16)] = -(dsv * ddv) * w_v[pl.ds(o, 16)]
            return carry

        lax.fori_loop(0, NCHG, chunk, 0)
        pltpu.sync_copy(nrm_v, out.at[w])

    return k


@functools.lru_cache(maxsize=None)
def _lap_kernel(N, NCHG):
    rows = N // NS
    NCHH = NCHG // 2

    @functools.partial(
        pl.kernel,
        out_type=jax.ShapeDtypeStruct((NC, N, LANE), F32),
        mesh=_mesh(),
        compiler_params=pltpu.CompilerParams(needs_layout_passes=False),
        scratch_types=[
            pltpu.VMEM((NCHH, LANE), jnp.int32),
            pltpu.VMEM((NCHH, LANE), jnp.int32),
            pltpu.VMEM((NCHH * LANE,), F32),
            pltpu.VMEM((LANE, LANE), F32),
            pltpu.VMEM((LANE, LANE), F32),
            pltpu.VMEM_SHARED((N, LANE), F32),
            pltpu.SemaphoreType.DMA,
            pltpu.SemaphoreType.DMA,
            pltpu.SemaphoreType.DMA,
            pltpu.SemaphoreType.DMA,
        ],
    )
    def k(v, srcr, dstr, normf, z128, out, src_v, dst_v, nrm_v,
          bf0, bf1, acc, g0, g1, s0, s1):
        bufs = [bf0, bf1]
        gs = [g0, g1]
        ss = [s0, s1]
        c = lax.axis_index("c")
        s = lax.axis_index("s")
        w = c * NS + s
        pltpu.sync_copy(z128.at[pl.ds(s * rows, rows)],
                        acc.at[pl.ds(s * rows, rows)])
        iota = lax.iota(jnp.int32, 16)
        cols = [r * 16 + iota for r in range(8)]
        plsc.subcore_barrier()

        def g_start(j, b):
            pltpu.async_copy(v.at[src_v.at[j]], bufs[b], gs[b])

        def g_wait(j, b):
            pltpu.make_async_copy(v.at[src_v.at[j]], bufs[b], gs[b]).wait()

        def s_start(j, b):
            pltpu.async_copy(bufs[b], acc.at[dst_v.at[j]], ss[b], add=True)

        def s_wait(j, b):
            pltpu.make_async_copy(bufs[b], acc.at[dst_v.at[j]], ss[b]).wait()

        def scale(j, b):
            buf = bufs[b]
            jbase = j * LANE

            def grp(g, c2):
                for l in range(16):
                    i = g * 16 + l
                    nv = plsc.load_gather(
                        nrm_v, [jnp.full((16,), jbase + i, jnp.int32)])
                    row = jnp.full((16,), i, jnp.int32)
                    for r in range(8):
                        x = plsc.load_gather(buf, [row, cols[r]])
                        plsc.store_scatter(buf, [row, cols[r]], x * nv)
                return c2

            lax.fori_loop(0, 8, grp, 0)

        def full_step(cc, b):
            s_wait(cc - 1, 1 - b)
            g_start(cc + 1, 1 - b)
            g_wait(cc, b)
            scale(cc, b)
            s_start(cc, b)

        for half in range(2):
            pltpu.sync_copy(srcr.at[w, pl.ds(half * NCHH, NCHH)], src_v)
            pltpu.sync_copy(dstr.at[w, pl.ds(half * NCHH, NCHH)], dst_v)
            pltpu.sync_copy(
                normf.at[w, pl.ds(half * NCHH * LANE, NCHH * LANE)], nrm_v)
            g_start(0, 0)
            g_start(1, 1)
            g_wait(0, 0)
            scale(0, 0)
            s_start(0, 0)
            full_step(1, 1)

            def pairs(p, carry):
                cc = 2 + p * 2
                full_step(cc, 0)
                full_step(cc + 1, 1)
                return carry

            lax.fori_loop(0, (NCHH - 4) // 2, pairs, 0)
            full_step(NCHH - 2, 0)
            g_wait(NCHH - 1, 1)
            scale(NCHH - 1, 1)
            s_start(NCHH - 1, 1)
            s_wait(NCHH - 2, 0)
            s_wait(NCHH - 1, 1)
        plsc.subcore_barrier()
        pltpu.sync_copy(acc.at[pl.ds(s * rows, rows)],
                        out.at[c, pl.ds(s * rows, rows)])

    return k


def _tc_dis(degp):
    N = degp.shape[1]

    def body(d_ref, o_ref):
        deg = jnp.sum(d_ref[:, :, 0], axis=0)
        safe = jnp.where(deg > 0, deg, 1.0)
        o_ref[...] = jnp.where(deg > 0, lax.rsqrt(safe), 0.0)[None, :]

    return pl.pallas_call(
        body, out_shape=jax.ShapeDtypeStruct((1, N), F32))(degp)


def _tc_add2(a, b, mb=1264):
    N, D = a.shape

    def body(a_ref, b_ref, o_ref):
        o_ref[...] = a_ref[...] + b_ref[...]

    spec = pl.BlockSpec((mb, D), lambda i: (i, 0))
    return pl.pallas_call(
        body, grid=(N // mb,), in_specs=[spec, spec], out_specs=spec,
        out_shape=jax.ShapeDtypeStruct((N, D), F32))(a, b)


def _tc_xg(x, l1, q0, q1, w, bias, mb=1264):
    N, D = x.shape
    Do = w.shape[-1]

    def body(x_r, l1_r, q0_r, q1_r, w_r, b_r, o_r):
        acc = jnp.dot(x_r[...], w_r[0], preferred_element_type=F32)
        acc += jnp.dot(l1_r[...], w_r[1], preferred_element_type=F32)
        acc += jnp.dot(q0_r[...] + q1_r[...], w_r[2],
                       preferred_element_type=F32)
        o_r[...] = acc + b_r[...]

    sp = pl.BlockSpec((mb, D), lambda i: (i, 0))
    return pl.pallas_call(
        body, grid=(N // mb,),
        in_specs=[sp, sp, sp, sp,
                  pl.BlockSpec((3, D, Do), lambda i: (0, 0, 0)),
                  pl.BlockSpec((1, Do), lambda i: (0, 0))],
        out_specs=pl.BlockSpec((mb, Do), lambda i: (i, 0)),
        out_shape=jax.ShapeDtypeStruct((N, Do), F32))(x, l1, q0, q1, w, bias)


def _tc_gates(h, h1, q0, q1, w, bias, xzr, mb=1264):
    N, D = h.shape
    Do = w.shape[-1]

    def body(h_r, h1_r, q0_r, q1_r, w_r, b_r, x_r, z_o, hr_o):
        acc = jnp.dot(h_r[...], w_r[0], preferred_element_type=F32)
        acc += jnp.dot(h1_r[...], w_r[1], preferred_element_type=F32)
        acc += jnp.dot(q0_r[...] + q1_r[...], w_r[2],
                       preferred_element_type=F32)
        zr = jax.nn.sigmoid(acc + b_r[...] + x_r[...])
        z_o[...] = zr[:, :D]
        hr_o[...] = h_r[...] * zr[:, D:]

    sp = pl.BlockSpec((mb, D), lambda i: (i, 0))
    so = pl.BlockSpec((mb, Do), lambda i: (i, 0))
    return pl.pallas_call(
        body, grid=(N // mb,),
        in_specs=[sp, sp, sp, sp,
                  pl.BlockSpec((3, D, Do), lambda i: (0, 0, 0)),
                  pl.BlockSpec((1, Do), lambda i: (0, 0)), so],
        out_specs=[sp, sp],
        out_shape=[jax.ShapeDtypeStruct((N, D), F32),
                   jax.ShapeDtypeStruct((N, D), F32)])(
                       h, h1, q0, q1, w, bias, xzr)


def _tc_update(hr, hr1, s0, s1, w, bias, xh, z, h, mb=1264):
    N, D = hr.shape

    def body(hr_r, hr1_r, s0_r, s1_r, w_r, b_r, x_r, z_r, h_r, o_r):
        acc = jnp.dot(hr_r[...], w_r[0], preferred_element_type=F32)
        acc += jnp.dot(hr1_r[...], w_r[1], preferred_element_type=F32)
        acc += jnp.dot(s0_r[...] + s1_r[...], w_r[2],
                       preferred_element_type=F32)
        ht = jnp.tanh(acc + b_r[...] + x_r[...])
        z = z_r[...]
        o_r[...] = jnp.maximum(z * h_r[...] + (1.0 - z) * ht, 0.0)

    sp = pl.BlockSpec((mb, D), lambda i: (i, 0))
    return pl.pallas_call(
        body, grid=(N // mb,),
        in_specs=[sp, sp, sp, sp,
                  pl.BlockSpec((3, D, D), lambda i: (0, 0, 0)),
                  pl.BlockSpec((1, D), lambda i: (0, 0)), sp, sp, sp],
        out_specs=sp,
        out_shape=jax.ShapeDtypeStruct((N, D), F32))(
            hr, hr1, s0, s1, w, bias, xh, z, h)


def _tc_h0(xg, bzr, bht, mb=1264):
    N = xg.shape[0]
    D = bht.shape[-1]

    def body(x_r, bz_r, bh_r, o_r):
        z = jax.nn.sigmoid(x_r[:, :D] + bz_r[:, :D])
        ht = jnp.tanh(x_r[:, 2 * D:] + bh_r[...])
        o_r[...] = jnp.maximum((1.0 - z) * ht, 0.0)

    return pl.pallas_call(
        body, grid=(N // mb,),
        in_specs=[pl.BlockSpec((mb, 3 * D), lambda i: (i, 0)),
                  pl.BlockSpec((1, 2 * D), lambda i: (0, 0)),
                  pl.BlockSpec((1, D), lambda i: (0, 0))],
        out_specs=pl.BlockSpec((mb, D), lambda i: (i, 0)),
        out_shape=jax.ShapeDtypeStruct((N, D), F32))(xg, bzr, bht)


def _tc_mm_relu(x, w, bias, bn, bk, relu=True):
    M, K = x.shape
    Nout = w.shape[1]
    gk = K // bk

    def body(x_r, w_r, b_r, o_r, acc):
        @pl.when(pl.program_id(1) == 0)
        def _():
            acc[...] = jnp.zeros_like(acc)

        acc[...] += jnp.dot(x_r[...], w_r[...], preferred_element_type=F32)

        @pl.when(pl.program_id(1) == gk - 1)
        def _():
            r = acc[...] + b_r[...]
            o_r[...] = jnp.maximum(r, 0.0) if relu else r

    return pl.pallas_call(
        body, grid=(Nout // bn, gk),
        in_specs=[pl.BlockSpec((M, bk), lambda n, k: (0, k)),
                  pl.BlockSpec((bk, bn), lambda n, k: (k, n)),
                  pl.BlockSpec((1, bn), lambda n, k: (0, n))],
        out_specs=pl.BlockSpec((M, bn), lambda n, k: (0, n)),
        scratch_shapes=[pltpu.VMEM((M, bn), F32)],
        out_shape=jax.ShapeDtypeStruct((M, Nout), F32))(x, w, bias)


def _tc_head3(x, w, bias):
    M, K = x.shape

    def body(x_r, w_r, b_r, o_r):
        logits = jnp.dot(x_r[...], w_r[...], preferred_element_type=F32)
        p = jax.nn.softmax(logits + b_r[...], axis=-1)
        o_r[...] = p[:, :10]

    return pl.pallas_call(
        body, out_shape=jax.ShapeDtypeStruct((M, 10), F32))(x, w, bias)


def kernel(x_temporal, edge_weight, Wx, bx, Wh, bh, W1, b1, W2, b2, W3, b3,
           edge_index, batch):
    T, N, D = x_temporal.shape
    E = edge_weight.shape[0]
    NCH0 = -(-E // (NW * LANE))
    NCHG = 4 * (-(-NCH0 // 4))
    EP = NW * NCHG * LANE
    NP = LANE * (-(-N // LANE))

    src = jnp.pad(edge_index[0].astype(jnp.int32), (0, EP - E))
    dst = jnp.pad(edge_index[1].astype(jnp.int32), (0, EP - E))
    ewp = jnp.pad(edge_weight.astype(F32), (0, EP - E))
    srcr = src.reshape(NW, NCHG, LANE)
    dstr = dst.reshape(NW, NCHG, LANE)
    xp = jnp.pad(x_temporal, ((0, 0), (0, NP - N), (0, 0)))
    z128 = jnp.zeros((NP, LANE), F32)

    srcf = src.reshape(NW, NCHG * LANE)
    dstf = dst.reshape(NW, NCHG * LANE)
    ewf = ewp.reshape(NW, NCHG * LANE)
    degp = _deg_kernel(NP, NCHG)(srcr, ewf, z128)
    dis = _tc_dis(degp).reshape(NP)
    normf = _norm_kernel(NP, NCHG)(srcf, dstf, ewf, dis)

    lap = _lap_kernel(NP, NCHG)

    def lap_parts(v):
        p = lap(v, srcr, dstr, normf, z128)
        return p[0], p[1]

    def prep(wstack, gates):
        cat = [jnp.concatenate([wstack[g, k] for g in gates], axis=1)
               for k in range(3)]
        return jnp.stack([cat[0] - cat[2], cat[1], 2.0 * cat[2]])

    WX = prep(Wx, (0, 1, 2))
    bxc = jnp.concatenate([bx[0], bx[1], bx[2]]).reshape(1, 3 * D)
    WZR = prep(Wh, (0, 1))
    bzr = jnp.concatenate([bh[0], bh[1]]).reshape(1, 2 * D)
    WHT = prep(Wh, (2,))
    bht = bh[2].reshape(1, D)

    XG = []
    for t in range(T):
        xt = xp[t]
        p0, p1 = lap_parts(xt)
        l1 = _tc_add2(p0, p1)
        q0, q1 = lap_parts(l1)
        XG.append(_tc_xg(xt, l1, q0, q1, WX, bxc))

    H = _tc_h0(XG[0], bzr, bht)
    for t in range(1, T):
        p0, p1 = lap_parts(H)
        h1 = _tc_add2(p0, p1)
        q0, q1 = lap_parts(h1)
        Z, HR = _tc_gates(H, h1, q0, q1, WZR, bzr, XG[t][:, :2 * D])
        r0, r1 = lap_parts(HR)
        hr1 = _tc_add2(r0, r1)
        s0, s1 = lap_parts(hr1)
        H = _tc_update(HR, hr1, s0, s1, WHT, bht, XG[t][:, 2 * D:], Z, H)

    numNodes = 100
    B = N // numNodes
    x = H[:N].reshape(B, numNodes * D)
    y1 = _tc_mm_relu(x, W1, b1.reshape(1, -1), bn=1280, bk=2560)
    y2 = _tc_mm_relu(y1, W2, b2.reshape(1, -1), bn=640, bk=1280)
    w3p = jnp.pad(W3, ((0, 0), (0, 118)))
    b3p = jnp.concatenate([b3, jnp.full((118,), -1e30, F32)]).reshape(1, 128)
    return _tc_head3(y2, w3p, b3p)
